# baseline (speedup 1.0000x reference)
import functools

import jax
import jax.numpy as jnp
from jax.experimental import pallas as pl
from jax.experimental.pallas import tpu as pltpu

_NCH = 8


def _copy_body(src, dst, sems):
    n_rows = src.shape[0] // _NCH
    cps = [
        pltpu.make_async_copy(
            src.at[pl.ds(i * n_rows, n_rows)],
            dst.at[pl.ds(i * n_rows, n_rows)],
            sems.at[i],
        )
        for i in range(_NCH)
    ]
    for c in cps:
        c.start()
    for c in cps:
        c.wait()


def kernel(x, emb_weight):
    seq_len = x.shape[1]
    dim = emb_weight.shape[1]
    return pl.pallas_call(
        _copy_body,
        in_specs=[pl.BlockSpec(memory_space=pltpu.MemorySpace.HBM)],
        out_specs=pl.BlockSpec(memory_space=pltpu.MemorySpace.HBM),
        out_shape=jax.ShapeDtypeStruct((seq_len, dim), jnp.float32),
        scratch_shapes=[pltpu.SemaphoreType.DMA((_NCH,))],
    )(emb_weight)


# speedup vs baseline: 37.9323x; 37.9323x over previous
import jax
import jax.numpy as jnp
from jax.experimental import pallas as pl
from jax.experimental.pallas import tpu as pltpu

_BLK = 512


def _copy_body(src, dst):
    dst[...] = src[...]


def kernel(x, emb_weight):
    seq_len = x.shape[1]
    dim = emb_weight.shape[1]
    return pl.pallas_call(
        _copy_body,
        grid=(seq_len // _BLK,),
        in_specs=[pl.BlockSpec((_BLK, dim), lambda i: (i, 0))],
        out_specs=pl.BlockSpec((_BLK, dim), lambda i: (i, 0)),
        out_shape=jax.ShapeDtypeStruct((seq_len, dim), jnp.float32),
    )(emb_weight)
